# baseline (speedup 1.0000x reference)
import jax
import jax.numpy as jnp
from jax.experimental import pallas as pl
from jax.experimental.pallas import tpu as pltpu

_E = 8
_TOPK = 2
_CHUNK = 1024


def _routing(logits):
    m1 = jnp.max(logits, axis=-1, keepdims=True)
    i1 = jnp.argmax(logits, axis=-1)
    iota = jax.lax.broadcasted_iota(jnp.int32, logits.shape, 1)
    masked = jnp.where(iota == i1[:, None], -jnp.inf, logits)
    m2 = jnp.max(masked, axis=-1, keepdims=True)
    i2 = jnp.argmax(masked, axis=-1)
    w1 = 1.0 / (1.0 + jnp.exp(m2 - m1))
    vals = jnp.concatenate([w1, 1.0 - w1], axis=1)
    idx = jnp.concatenate([i1[:, None], i2[:, None]], axis=1)
    return vals, idx


def _pipeline_body(x_ref, w_ref, b_ref, logits_ref, vals_ref, idx_ref):
    logits = jax.lax.dot_general(
        x_ref[...], w_ref[...], (((1,), (1,)), ((), ())),
        preferred_element_type=jnp.float32,
    ) + b_ref[...]
    logits_ref[...] = logits
    vals, idx = _routing(logits)
    vals_ref[...] = vals
    idx_ref[...] = idx


def kernel(hidden_states, W, b):
    orig_shape = hidden_states.shape
    x = hidden_states.reshape(-1, orig_shape[-1])
    n_tokens, hidden = x.shape
    n_chunks = n_tokens // _CHUNK
    mesh = pltpu.create_tensorcore_mesh("core")

    def run(refs):
        x_ref, w_ref, b_ref, logits_ref, vals_ref, idx_ref = refs

        @pl.core_map(mesh)
        def _per_core():
            core = jax.lax.axis_index("core")
            n_cores = jax.lax.axis_size("core")
            n_per_core = n_chunks // n_cores
            base = core * n_per_core
            pipeline = pltpu.emit_pipeline(
                _pipeline_body,
                grid=(n_per_core,),
                in_specs=[
                    pl.BlockSpec((_CHUNK, hidden), lambda i: (base + i, 0)),
                    pl.BlockSpec((_E, hidden), lambda i: (0, 0)),
                    pl.BlockSpec((1, _E), lambda i: (0, 0)),
                ],
                out_specs=[
                    pl.BlockSpec((_CHUNK, _E), lambda i: (base + i, 0)),
                    pl.BlockSpec((_CHUNK, _TOPK), lambda i: (base + i, 0)),
                    pl.BlockSpec((_CHUNK, _TOPK), lambda i: (base + i, 0)),
                ],
            )
            pipeline(x_ref, w_ref, b_ref, logits_ref, vals_ref, idx_ref)

    _, _, _, logits, vals, idx = pl.run_state(run)(
        (
            x,
            W,
            b.reshape(1, _E),
            jnp.zeros((n_tokens, _E), jnp.float32),
            jnp.zeros((n_tokens, _TOPK), jnp.float32),
            jnp.zeros((n_tokens, _TOPK), jnp.int32),
        )
    )
    return (logits, vals, idx)


# speedup vs baseline: 1.3718x; 1.3718x over previous
import jax
import jax.numpy as jnp
from jax.experimental import pallas as pl
from jax.experimental.pallas import tpu as pltpu

_E = 8
_TOPK = 2
_BR = 4096
_NSPLIT = 4
_SUB = _BR // _NSPLIT


def _routing(logits):
    m1 = jnp.max(logits, axis=-1, keepdims=True)
    i1 = jnp.argmax(logits, axis=-1)
    iota = jax.lax.broadcasted_iota(jnp.int32, logits.shape, 1)
    masked = jnp.where(iota == i1[:, None], -jnp.inf, logits)
    m2 = jnp.max(masked, axis=-1, keepdims=True)
    i2 = jnp.argmax(masked, axis=-1)
    w1 = 1.0 / (1.0 + jnp.exp(m2 - m1))
    vals = jnp.concatenate([w1, 1.0 - w1], axis=1)
    idx = jnp.concatenate([i1[:, None], i2[:, None]], axis=1)
    return vals, idx


def _router_block(*refs):
    x_refs = refs[:_NSPLIT]
    w_ref, b_ref, logits_ref, vals_ref, idx_ref = refs[_NSPLIT:]
    w = w_ref[...]
    b = b_ref[...]
    for j in range(_NSPLIT):
        logits = jax.lax.dot_general(
            x_refs[j][...], w, (((1,), (1,)), ((), ())),
            preferred_element_type=jnp.float32,
        ) + b
        logits_ref[pl.ds(j * _SUB, _SUB), :] = logits
        vals, idx = _routing(logits)
        vals_ref[pl.ds(j * _SUB, _SUB), :] = vals
        idx_ref[pl.ds(j * _SUB, _SUB), :] = idx


def _make_x_spec(j):
    return pl.BlockSpec((_SUB, 768), lambda i, j=j: (i * _NSPLIT + j, 0))


def kernel(hidden_states, W, b):
    orig_shape = hidden_states.shape
    x = hidden_states.reshape(-1, orig_shape[-1])
    n_tokens, hidden = x.shape
    grid = (n_tokens // _BR,)

    logits, vals, idx = pl.pallas_call(
        _router_block,
        grid=grid,
        in_specs=[_make_x_spec(j) for j in range(_NSPLIT)]
        + [
            pl.BlockSpec((_E, hidden), lambda i: (0, 0)),
            pl.BlockSpec((1, _E), lambda i: (0, 0)),
        ],
        out_specs=[
            pl.BlockSpec((_BR, _E), lambda i: (i, 0)),
            pl.BlockSpec((_BR, _TOPK), lambda i: (i, 0)),
            pl.BlockSpec((_BR, _TOPK), lambda i: (i, 0)),
        ],
        out_shape=[
            jax.ShapeDtypeStruct((n_tokens, _E), jnp.float32),
            jax.ShapeDtypeStruct((n_tokens, _TOPK), jnp.float32),
            jax.ShapeDtypeStruct((n_tokens, _TOPK), jnp.int32),
        ],
        compiler_params=pltpu.CompilerParams(
            dimension_semantics=("arbitrary",),
        ),
    )(x, x, x, x, W, b.reshape(1, _E))

    return (logits, vals, idx)


# speedup vs baseline: 2.6751x; 1.9501x over previous
import jax
import jax.numpy as jnp
from jax.experimental import pallas as pl
from jax.experimental.pallas import tpu as pltpu

_E = 8
_TOPK = 2
_BR = 4096


def _router_block(x_ref, w_ref, b_ref, logits_ref, vals_ref, idx_ref):
    x = x_ref[...]
    w = w_ref[...]
    logits = jax.lax.dot_general(
        x, w, (((1,), (1,)), ((), ())), preferred_element_type=jnp.float32
    ) + b_ref[...]
    logits_ref[...] = logits.T

    m1 = jnp.max(logits, axis=-1, keepdims=True)
    i1 = jnp.argmax(logits, axis=-1)
    iota = jax.lax.broadcasted_iota(jnp.int32, logits.shape, 1)
    masked = jnp.where(iota == i1[:, None], -jnp.inf, logits)
    m2 = jnp.max(masked, axis=-1, keepdims=True)
    i2 = jnp.argmax(masked, axis=-1)
    w1 = 1.0 / (1.0 + jnp.exp(m2 - m1))
    vals_ref[...] = jnp.concatenate([w1.T, 1.0 - w1.T], axis=0)
    idx_ref[...] = jnp.concatenate([i1[None, :], i2[None, :]], axis=0)


def kernel(hidden_states, W, b):
    orig_shape = hidden_states.shape
    x = hidden_states.reshape(-1, orig_shape[-1])
    n_tokens, hidden = x.shape
    grid = (n_tokens // _BR,)

    logits_t, vals_t, idx_t = pl.pallas_call(
        _router_block,
        grid=grid,
        in_specs=[
            pl.BlockSpec((_BR, hidden), lambda i: (i, 0)),
            pl.BlockSpec((_E, hidden), lambda i: (0, 0)),
            pl.BlockSpec((1, _E), lambda i: (0, 0)),
        ],
        out_specs=[
            pl.BlockSpec((_E, _BR), lambda i: (0, i)),
            pl.BlockSpec((_TOPK, _BR), lambda i: (0, i)),
            pl.BlockSpec((_TOPK, _BR), lambda i: (0, i)),
        ],
        out_shape=[
            jax.ShapeDtypeStruct((_E, n_tokens), jnp.float32),
            jax.ShapeDtypeStruct((_TOPK, n_tokens), jnp.float32),
            jax.ShapeDtypeStruct((_TOPK, n_tokens), jnp.int32),
        ],
        compiler_params=pltpu.CompilerParams(
            dimension_semantics=("arbitrary",),
        ),
    )(x, W, b.reshape(1, _E))

    return (logits_t.T, vals_t.T, idx_t.T)
